# baseline (speedup 1.0000x reference)
import functools

import jax
import jax.numpy as jnp
from jax import lax
from jax.experimental import pallas as pl
from jax.experimental.pallas import tpu as pltpu
from jax.experimental.pallas import tpu_sc as plsc

NC = 2
NS = 16
L = 16

CHUNK = 128
N_CHUNKS = 80
UNROLL = 4


def _sc_propagate(x, src3, dst3, w3):
    n, d = x.shape
    npad = ((n + 8 * NS - 1) // (8 * NS)) * (8 * NS)
    rows_per_tile = npad // NS
    half = N_CHUNKS // 2

    mesh = plsc.VectorSubcoreMesh(
        core_axis_name="c", subcore_axis_name="s", num_cores=NC, num_subcores=NS
    )

    @functools.partial(
        pl.kernel,
        out_type=jax.ShapeDtypeStruct((NC * npad, d), jnp.float32),
        mesh=mesh,
        scratch_types=[
            pltpu.VMEM((half, CHUNK), jnp.int32),
            pltpu.VMEM((half, CHUNK), jnp.int32),
            pltpu.VMEM((half, CHUNK), jnp.float32),
            pltpu.VMEM((CHUNK, d), jnp.float32),
            pltpu.VMEM((CHUNK, d), jnp.float32),
            pltpu.VMEM_SHARED((npad, d), jnp.float32),
            pltpu.SemaphoreType.DMA,
            pltpu.SemaphoreType.DMA,
            pltpu.SemaphoreType.DMA,
            pltpu.SemaphoreType.DMA,
        ],
        compiler_params=pltpu.CompilerParams(needs_layout_passes=False),
    )
    def k(x_hbm, src_hbm, dst_hbm, w_hbm, out_hbm,
          sidx, didx, wv, rows0, rows1, acc, gsem0, gsem1, ssem0, ssem1):
        c = lax.axis_index("c")
        s = lax.axis_index("s")
        wid = c * NS + s

        def zrow(r, _):
            for kk in range(d // L):
                rows0[r, pl.ds(kk * L, L)] = jnp.zeros((L,), jnp.float32)
            return 0

        lax.fori_loop(0, CHUNK, zrow, 0)
        row0 = s * rows_per_tile
        for j in range(rows_per_tile // CHUNK):
            pltpu.sync_copy(rows0, acc.at[pl.ds(row0 + j * CHUNK, CHUNK)])
        plsc.subcore_barrier()

        def gstart(i, buf, sem):
            return pltpu.async_copy(x_hbm.at[sidx.at[i]], buf, sem)

        def gwait(i, buf, sem):
            pltpu.make_async_copy(x_hbm.at[sidx.at[i]], buf, sem).wait()

        def scale(i, buf):
            def body(u, _):
                for t in range(UNROLL):
                    e = u * UNROLL + t
                    wsplat = plsc.load_gather(
                        wv, [jnp.full((L,), i, jnp.int32), jnp.full((L,), e, jnp.int32)]
                    )
                    for kk in range(d // L):
                        sl = pl.ds(kk * L, L)
                        buf[e, sl] = buf[e, sl] * wsplat
                return 0

            lax.fori_loop(0, CHUNK // UNROLL, body, 0)

        def sstart(i, buf, sem):
            return pltpu.async_copy(buf, acc.at[didx.at[i]], sem, add=True)

        for h in range(2):
            pltpu.sync_copy(src_hbm.at[wid, pl.ds(h * half, half)], sidx)
            pltpu.sync_copy(dst_hbm.at[wid, pl.ds(h * half, half)], didx)
            pltpu.sync_copy(w_hbm.at[wid, pl.ds(h * half, half)], wv)
            gstart(0, rows0, gsem0)

            def pair_body(j, _):
                i0 = 2 * j
                i1 = 2 * j + 1
                i2 = lax.rem(2 * j + 2, half)
                gwait(i0, rows0, gsem0)
                g1 = gstart(i1, rows1, gsem1)
                scale(i0, rows0)
                s0 = sstart(i0, rows0, ssem0)
                g1.wait()
                s0.wait()
                gstart(i2, rows0, gsem0)
                scale(i1, rows1)
                s1 = sstart(i1, rows1, ssem1)
                s1.wait()
                return 0

            lax.fori_loop(0, half // 2, pair_body, 0)
            gwait(0, rows0, gsem0)
        plsc.subcore_barrier()

        pltpu.sync_copy(
            acc.at[pl.ds(row0, rows_per_tile)],
            out_hbm.at[pl.ds(c * npad + row0, rows_per_tile)],
        )

    return k(x, src3, dst3, w3)


def _tc_dense(p0, p1, x, w1, b1, w2, b2):
    n, d = x.shape
    bm = 2000

    def body(p0_ref, p1_ref, x_ref, w1_ref, b1_ref, w2_ref, b2_ref, o_ref):
        xp = p0_ref[...] + p1_ref[...]
        h1 = xp + x_ref[...]
        h2 = xp * x_ref[...]
        dn = (((1,), (1,)), ((), ()))
        acc = lax.dot_general(h1, w1_ref[...], dn, preferred_element_type=jnp.float32)
        acc = acc + lax.dot_general(h2, w2_ref[...], dn, preferred_element_type=jnp.float32)
        acc = acc + b1_ref[...] + b2_ref[...]
        o_ref[...] = jnp.maximum(acc, 0.0)

    row_spec = pl.BlockSpec((bm, d), lambda i: (i, 0))
    full_spec = pl.BlockSpec((d, d), lambda i: (0, 0))
    bias_spec = pl.BlockSpec((1, d), lambda i: (0, 0))
    return pl.pallas_call(
        body,
        out_shape=jax.ShapeDtypeStruct((n, d), jnp.float32),
        grid=(n // bm,),
        in_specs=[row_spec, row_spec, row_spec, full_spec, bias_spec, full_spec, bias_spec],
        out_specs=row_spec,
    )(p0, p1, x, w1, b1.reshape(1, d), w2, b2.reshape(1, d))


def kernel(x, edge_index, edge_weight, W1, b1, W2, b2):
    n, d = x.shape
    e = edge_weight.shape[0]
    nw = NC * NS
    e_pad = nw * N_CHUNKS * CHUNK
    src = edge_index[0].astype(jnp.int32)
    dst = edge_index[1].astype(jnp.int32)
    w = edge_weight.astype(jnp.float32)
    pad = e_pad - e
    src3 = jnp.pad(src, (0, pad)).reshape(nw, N_CHUNKS, CHUNK)
    dst3 = jnp.pad(dst, (0, pad)).reshape(nw, N_CHUNKS, CHUNK)
    w3 = jnp.pad(w, (0, pad)).reshape(nw, N_CHUNKS, CHUNK)
    pflat = _sc_propagate(x, src3, dst3, w3)
    npad = pflat.shape[0] // NC
    return _tc_dense(pflat[:n], pflat[npad:npad + n], x, W1, b1, W2, b2)


# speedup vs baseline: 2.5534x; 2.5534x over previous
import functools

import jax
import jax.numpy as jnp
from jax import lax
from jax.experimental import pallas as pl
from jax.experimental.pallas import tpu as pltpu
from jax.experimental.pallas import tpu_sc as plsc

NC = 2
NS = 16
L = 16

CHUNK = 128
N_CHUNKS = 80
UNROLL = 4


def _sc_propagate(x, src3, dst3, w3):
    n, d = x.shape
    npad = ((n + 8 * NS - 1) // (8 * NS)) * (8 * NS)
    rows_per_tile = npad // NS
    half = N_CHUNKS // 2

    mesh = plsc.VectorSubcoreMesh(
        core_axis_name="c", subcore_axis_name="s", num_cores=NC, num_subcores=NS
    )

    @functools.partial(
        pl.kernel,
        out_type=jax.ShapeDtypeStruct((NC * npad, d), jnp.float32),
        mesh=mesh,
        scratch_types=[
            pltpu.VMEM((half, CHUNK), jnp.int32),
            pltpu.VMEM((half, CHUNK), jnp.int32),
            pltpu.VMEM((half, CHUNK), jnp.float32),
            pltpu.VMEM((CHUNK, d), jnp.float32),
            pltpu.VMEM((CHUNK, d), jnp.float32),
            pltpu.VMEM_SHARED((npad, d), jnp.float32),
            pltpu.SemaphoreType.DMA,
            pltpu.SemaphoreType.DMA,
            pltpu.SemaphoreType.DMA,
            pltpu.SemaphoreType.DMA,
        ],
        compiler_params=pltpu.CompilerParams(needs_layout_passes=False),
    )
    def k(x_hbm, src_hbm, dst_hbm, w_hbm, out_hbm,
          sidx, didx, wv, rows0, rows1, acc, gsem0, gsem1, ssem0, ssem1):
        c = lax.axis_index("c")
        s = lax.axis_index("s")
        wid = c * NS + s

        def zrow(r, _):
            for kk in range(d // L):
                rows0[r, pl.ds(kk * L, L)] = jnp.zeros((L,), jnp.float32)
            return 0

        lax.fori_loop(0, CHUNK, zrow, 0)
        row0 = s * rows_per_tile
        for j in range(rows_per_tile // CHUNK):
            pltpu.sync_copy(rows0, acc.at[pl.ds(row0 + j * CHUNK, CHUNK)])
        plsc.subcore_barrier()

        def gstart(i, buf, sem):
            return pltpu.async_copy(x_hbm.at[sidx.at[i]], buf, sem)

        def gwait(i, buf, sem):
            pltpu.make_async_copy(x_hbm.at[sidx.at[i]], buf, sem).wait()

        def scale(i, buf):
            def body(u, _):
                for t in range(UNROLL):
                    e = u * UNROLL + t
                    wsplat = plsc.load_gather(
                        wv, [jnp.full((L,), i, jnp.int32), jnp.full((L,), e, jnp.int32)]
                    )
                    for kk in range(d // L):
                        sl = pl.ds(kk * L, L)
                        buf[e, sl] = buf[e, sl] * wsplat
                return 0

            lax.fori_loop(0, CHUNK // UNROLL, body, 0)

        def sstart(i, buf, sem):
            return pltpu.async_copy(buf, acc.at[didx.at[i]], sem, add=True)

        for h in range(2):
            pltpu.sync_copy(src_hbm.at[wid, pl.ds(h * half, half)], sidx)
            pltpu.sync_copy(dst_hbm.at[wid, pl.ds(h * half, half)], didx)
            pltpu.sync_copy(w_hbm.at[wid, pl.ds(h * half, half)], wv)
            gstart(0, rows0, gsem0)

            def pair_body(j, _):
                i0 = 2 * j
                i1 = 2 * j + 1
                i2 = lax.rem(2 * j + 2, half)
                gwait(i0, rows0, gsem0)
                g1 = gstart(i1, rows1, gsem1)
                scale(i0, rows0)
                s0 = sstart(i0, rows0, ssem0)
                g1.wait()
                s0.wait()
                gstart(i2, rows0, gsem0)
                scale(i1, rows1)
                s1 = sstart(i1, rows1, ssem1)
                s1.wait()
                return 0

            lax.fori_loop(0, half // 2, pair_body, 0)
            gwait(0, rows0, gsem0)
        plsc.subcore_barrier()

        pltpu.sync_copy(
            acc.at[pl.ds(row0, rows_per_tile)],
            out_hbm.at[pl.ds(c * npad + row0, rows_per_tile)],
        )

    return k(x, src3, dst3, w3)


def _tc_dense(p0, p1, x, w1, b1, w2, b2):
    n, d = x.shape
    bm = 2000

    def body(p0_ref, p1_ref, x_ref, w1_ref, b1_ref, w2_ref, b2_ref, o_ref):
        xp = p0_ref[...] + p1_ref[...]
        h1 = xp + x_ref[...]
        h2 = xp * x_ref[...]
        dn = (((1,), (1,)), ((), ()))
        acc = lax.dot_general(h1, w1_ref[...], dn, preferred_element_type=jnp.float32)
        acc = acc + lax.dot_general(h2, w2_ref[...], dn, preferred_element_type=jnp.float32)
        acc = acc + b1_ref[...] + b2_ref[...]
        o_ref[...] = jnp.maximum(acc, 0.0)

    row_spec = pl.BlockSpec((bm, d), lambda i: (i, 0))
    full_spec = pl.BlockSpec((d, d), lambda i: (0, 0))
    bias_spec = pl.BlockSpec((1, d), lambda i: (0, 0))
    return pl.pallas_call(
        body,
        out_shape=jax.ShapeDtypeStruct((n, d), jnp.float32),
        grid=(n // bm,),
        in_specs=[row_spec, row_spec, row_spec, full_spec, bias_spec, full_spec, bias_spec],
        out_specs=row_spec,
    )(p0, p1, x, w1, b1.reshape(1, d), w2, b2.reshape(1, d))


def kernel(x, edge_index, edge_weight, W1, b1, W2, b2):
    n, d = x.shape
    e = edge_weight.shape[0]
    nw = NC * NS
    e_pad = nw * N_CHUNKS * CHUNK
    src = edge_index[0].astype(jnp.int32)
    dst = edge_index[1].astype(jnp.int32)
    w = edge_weight.astype(jnp.float32)
    pad = e_pad - e
    spread = (jnp.arange(pad, dtype=jnp.int32) * 8) % n
    src3 = jnp.concatenate([src, spread]).reshape(nw, N_CHUNKS, CHUNK)
    dst3 = jnp.concatenate([dst, spread]).reshape(nw, N_CHUNKS, CHUNK)
    w3 = jnp.pad(w, (0, pad)).reshape(nw, N_CHUNKS, CHUNK)
    pflat = _sc_propagate(x, src3, dst3, w3)
    npad = pflat.shape[0] // NC
    return _tc_dense(pflat[:n], pflat[npad:npad + n], x, W1, b1, W2, b2)
